# baseline (speedup 1.0000x reference)
import jax
import jax.numpy as jnp
from jax.experimental import pallas as pl
from jax.experimental.pallas import tpu as pltpu

VOCAB = 1000000
H = 64
CTX = 200
BLOCK_V = 25000
NB = VOCAB // BLOCK_V


def _summaries_body(ids_ref, idx_ref, numf_ref, tok_hbm, node_ref,
                    parent_ref, lang_ref, ts_out, fs_out,
                    tok_scratch, sem):
    def issue(t, _):
        pltpu.make_async_copy(
            tok_hbm.at[pl.ds(ids_ref[t], 1), :],
            tok_scratch.at[pl.ds(t, 1), :],
            sem,
        ).start()
        return 0
    jax.lax.fori_loop(0, CTX, issue, 0)

    def wait(t, _):
        pltpu.make_async_copy(
            tok_hbm.at[pl.ds(ids_ref[t], 1), :],
            tok_scratch.at[pl.ds(t, 1), :],
            sem,
        ).wait()
        return 0
    jax.lax.fori_loop(0, CTX, wait, 0)

    ts_out[...] = jnp.sum(tok_scratch[...], axis=0, keepdims=True) * (1.0 / CTX)
    node_row = node_ref[pl.ds(idx_ref[0], 1), :]
    parent_row = parent_ref[pl.ds(idx_ref[1], 1), :]
    lang_row = lang_ref[pl.ds(idx_ref[2], 1), :]
    fs_out[...] = node_row + parent_row + lang_row


def _matvec_body(hid_ref, prefW_ref, pref_out):
    col = jax.lax.dot_general(
        prefW_ref[...], hid_ref[...],
        (((1,), (0,)), ((), ())),
        preferred_element_type=jnp.float32)
    pref_out[0] = col


def kernel(prefix_ids, node_idx, parent_idx, lang_idx, numeric_features,
           token_table, node_table, parent_table, lang_table,
           num_W, num_b, hid_W, hid_b, sw_W, sw_b, pref_W, pref_b):
    ids = prefix_ids[-CTX:].astype(jnp.int32)
    idx3 = jnp.stack([jnp.asarray(node_idx, jnp.int32),
                      jnp.asarray(parent_idx, jnp.int32),
                      jnp.asarray(lang_idx, jnp.int32)])

    smem = pl.BlockSpec(memory_space=pltpu.MemorySpace.SMEM)
    vmem_full = pl.BlockSpec(memory_space=pltpu.MemorySpace.VMEM)

    ts, fs = pl.pallas_call(
        _summaries_body,
        in_specs=[
            smem,
            smem,
            smem,
            pl.BlockSpec(memory_space=pltpu.MemorySpace.HBM),
            vmem_full,
            vmem_full,
            vmem_full,
        ],
        out_specs=[vmem_full, vmem_full],
        out_shape=[
            jax.ShapeDtypeStruct((1, H), jnp.float32),
            jax.ShapeDtypeStruct((1, H), jnp.float32),
        ],
        scratch_shapes=[
            pltpu.VMEM((CTX, H), jnp.float32),
            pltpu.SemaphoreType.DMA,
        ],
    )(ids, idx3, numeric_features, token_table, node_table, parent_table,
      lang_table)

    token_summary = ts[0]
    feature_summary = fs[0] + (num_W @ numeric_features + num_b)
    hidden = jnp.tanh(
        hid_W @ jnp.concatenate([token_summary, feature_summary], axis=0)
        + hid_b)
    switch_logit = (sw_W @ hidden + sw_b)[0]

    pref = pl.pallas_call(
        _matvec_body,
        grid=(NB,),
        in_specs=[
            vmem_full,
            pl.BlockSpec((BLOCK_V, H), lambda i: (i, 0)),
        ],
        out_specs=pl.BlockSpec((1, BLOCK_V, 1), lambda i: (i, 0, 0)),
        out_shape=jax.ShapeDtypeStruct((NB, BLOCK_V, 1), jnp.float32),
    )(hidden.reshape(H, 1), pref_W)

    return (switch_logit, pref.reshape(VOCAB) + pref_b)


# speedup vs baseline: 1.3585x; 1.3585x over previous
import jax
import jax.numpy as jnp
from jax.experimental import pallas as pl
from jax.experimental.pallas import tpu as pltpu

VOCAB = 1000000
H = 64
CTX = 200
S = 4
B_EACH = 10000
NB = VOCAB // (S * B_EACH)


def _summaries_body(ids_ref, idx_ref, numf_ref, tok_hbm, node_ref,
                    parent_ref, lang_ref, ts_out, fs_out,
                    tok_scratch, sem):
    def issue(t, _):
        pltpu.make_async_copy(
            tok_hbm.at[pl.ds(ids_ref[t], 1), :],
            tok_scratch.at[pl.ds(t, 1), :],
            sem,
        ).start()
        return 0
    jax.lax.fori_loop(0, CTX, issue, 0)

    def wait(t, _):
        pltpu.make_async_copy(
            tok_hbm.at[pl.ds(ids_ref[t], 1), :],
            tok_scratch.at[pl.ds(t, 1), :],
            sem,
        ).wait()
        return 0
    jax.lax.fori_loop(0, CTX, wait, 0)

    ts_out[...] = jnp.sum(tok_scratch[...], axis=0, keepdims=True) * (1.0 / CTX)
    node_row = node_ref[pl.ds(idx_ref[0], 1), :]
    parent_row = parent_ref[pl.ds(idx_ref[1], 1), :]
    lang_row = lang_ref[pl.ds(idx_ref[2], 1), :]
    fs_out[...] = node_row + parent_row + lang_row


def _matvec_body(hid_ref, *refs):
    w_refs = refs[:S]
    out_refs = refs[S:]
    for k in range(S):
        out_refs[k][0] = jax.lax.dot_general(
            hid_ref[...], w_refs[k][...],
            (((1,), (1,)), ((), ())),
            preferred_element_type=jnp.float32)


def kernel(prefix_ids, node_idx, parent_idx, lang_idx, numeric_features,
           token_table, node_table, parent_table, lang_table,
           num_W, num_b, hid_W, hid_b, sw_W, sw_b, pref_W, pref_b):
    ids = prefix_ids[-CTX:].astype(jnp.int32)
    idx3 = jnp.stack([jnp.asarray(node_idx, jnp.int32),
                      jnp.asarray(parent_idx, jnp.int32),
                      jnp.asarray(lang_idx, jnp.int32)])

    smem = pl.BlockSpec(memory_space=pltpu.MemorySpace.SMEM)
    vmem_full = pl.BlockSpec(memory_space=pltpu.MemorySpace.VMEM)

    ts, fs = pl.pallas_call(
        _summaries_body,
        in_specs=[
            smem,
            smem,
            smem,
            pl.BlockSpec(memory_space=pltpu.MemorySpace.HBM),
            vmem_full,
            vmem_full,
            vmem_full,
        ],
        out_specs=[vmem_full, vmem_full],
        out_shape=[
            jax.ShapeDtypeStruct((1, H), jnp.float32),
            jax.ShapeDtypeStruct((1, H), jnp.float32),
        ],
        scratch_shapes=[
            pltpu.VMEM((CTX, H), jnp.float32),
            pltpu.SemaphoreType.DMA,
        ],
    )(ids, idx3, numeric_features, token_table, node_table, parent_table,
      lang_table)

    token_summary = ts[0]
    feature_summary = fs[0] + (num_W @ numeric_features + num_b)
    hidden = jnp.tanh(
        hid_W @ jnp.concatenate([token_summary, feature_summary], axis=0)
        + hid_b)
    switch_logit = (sw_W @ hidden + sw_b)[0]

    def w_spec(k):
        return pl.BlockSpec((B_EACH, H), lambda i, k=k: (S * i + k, 0))

    quarters = pl.pallas_call(
        _matvec_body,
        grid=(NB,),
        in_specs=[vmem_full] + [w_spec(k) for k in range(S)],
        out_specs=[pl.BlockSpec((1, 1, B_EACH), lambda i: (i, 0, 0))
                   for _ in range(S)],
        out_shape=[jax.ShapeDtypeStruct((NB, 1, B_EACH), jnp.float32)
                   for _ in range(S)],
    )(hidden.reshape(1, H), *([pref_W] * S))

    raw = jnp.stack([q.reshape(NB, B_EACH) for q in quarters], axis=1)
    return (switch_logit, raw.reshape(VOCAB) + pref_b)
